# Initial kernel scaffold; baseline (speedup 1.0000x reference)
import jax
import jax.numpy as jnp
from jax.experimental import pallas as pl


def kernel(hidden_states, Wq, bq, Wk, bk, Wv, bv, Wo, bo, Wg, bg):
    raise NotImplementedError("write your pallas kernel here")


# speedup vs baseline: 1.6720x; 1.6720x over previous
import functools

import jax
import jax.numpy as jnp
from jax import lax
from jax.experimental import pallas as pl
from jax.experimental.pallas import tpu as pltpu
from jax.experimental.pallas import tpu_sc as plsc

B_, L, H = 1, 2048, 2048
NH, DH = 16, 128
BS = 32
NB = L // BS
NSEL = 16
WIN = 256
QT = 256
NQT = L // QT
SCALE = 1.0 / (DH ** 0.5)
NEG = -1e9


def _proj_body(a_ref, b_ref, bias_ref, o_ref):
    acc = lax.dot_general(a_ref[...], b_ref[...], (((1,), (0,)), ((), ())),
                          preferred_element_type=jnp.float32)
    o_ref[...] = acc + bias_ref[0:1, :]


def _projection(hs, wall, ball):
    nw = wall.shape[1] // 128
    nj = 7 if nw % 7 == 0 else nw
    bn = (nw // nj) * 128
    bm = 512
    return pl.pallas_call(
        _proj_body,
        grid=(L // bm, nj),
        in_specs=[
            pl.BlockSpec((bm, H), lambda i, j: (i, 0)),
            pl.BlockSpec((H, bn), lambda i, j: (0, j)),
            pl.BlockSpec((8, bn), lambda i, j: (0, j)),
        ],
        out_specs=pl.BlockSpec((bm, bn), lambda i, j: (i, j)),
        out_shape=jax.ShapeDtypeStruct((L, nw * 128), jnp.float32),
        compiler_params=pltpu.CompilerParams(
            dimension_semantics=("parallel", "parallel")),
    )(hs, wall, jnp.broadcast_to(ball, (8, nw * 128)))


def _compress_body(q_ref, k_ref, v_ref, comp_ref, idx_ref, kh_ref, vh_ref):
    h = pl.program_id(0)
    q = q_ref[...]
    k = k_ref[...]
    v = v_ref[...]
    kh_ref[0] = k
    vh_ref[0] = v

    r64 = lax.broadcasted_iota(jnp.int32, (NB, L), 0)
    c64 = lax.broadcasted_iota(jnp.int32, (NB, L), 1)
    pool = jnp.where(c64 // BS == r64, jnp.float32(1.0 / BS), jnp.float32(0.0))
    ck = lax.dot_general(pool, k, (((1,), (0,)), ((), ())),
                         preferred_element_type=jnp.float32)
    cv = lax.dot_general(pool, v, (((1,), (0,)), ((), ())),
                         preferred_element_type=jnp.float32)

    s = lax.dot_general(q, ck, (((1,), (1,)), ((), ())),
                        preferred_element_type=jnp.float32) * SCALE
    m = jnp.max(s, axis=-1, keepdims=True)
    e = jnp.exp(s - m)
    p = e / jnp.sum(e, axis=-1, keepdims=True)
    comp_ref[...] = lax.dot_general(p, cv, (((1,), (0,)), ((), ())),
                                    preferred_element_type=jnp.float32)

    imp = jnp.sum(p, axis=0, keepdims=True)
    eye = jnp.where(lax.broadcasted_iota(jnp.int32, (NB, NB), 0)
                    == lax.broadcasted_iota(jnp.int32, (NB, NB), 1),
                    jnp.float32(1.0), jnp.float32(0.0))
    imp_c = lax.dot_general(eye, imp, (((1,), (1,)), ((), ())),
                            preferred_element_type=jnp.float32)

    gt = imp > imp_c
    tie = (imp == imp_c) & (c64[:, :NB] < r64[:, :NB])
    rank = jnp.sum((gt | tie).astype(jnp.float32), axis=1, keepdims=True)
    selm = rank < NSEL

    slot_i = lax.broadcasted_iota(jnp.int32, (NB, NSEL), 1).astype(jnp.float32)
    oh = ((rank == slot_i) & selm).astype(jnp.float32)
    r_i = lax.broadcasted_iota(jnp.int32, (NB, NSEL), 0).astype(jnp.float32)
    blk_row = jnp.sum(r_i * oh, axis=0, keepdims=True)

    pos = lax.broadcasted_iota(jnp.int32, (NSEL * BS, 1), 0)
    slot_of = pos // BS
    oh_pos = (lax.broadcasted_iota(jnp.int32, (NSEL * BS, NSEL), 1)
              == slot_of).astype(jnp.float32)
    blk_of = jnp.sum(oh_pos * blk_row, axis=1, keepdims=True)
    idx_ref[0] = (blk_of.astype(jnp.int32) * BS
                  + (pos - slot_of * BS) + h * L)


def _compress_select(y):
    return pl.pallas_call(
        _compress_body,
        grid=(NH,),
        in_specs=[
            pl.BlockSpec((L, DH), lambda h: (0, h)),
            pl.BlockSpec((L, DH), lambda h: (0, NH + h)),
            pl.BlockSpec((L, DH), lambda h: (0, 2 * NH + h)),
        ],
        out_specs=[
            pl.BlockSpec((L, DH), lambda h: (0, h)),
            pl.BlockSpec((1, NSEL * BS, 1), lambda h: (h, 0, 0)),
            pl.BlockSpec((1, L, DH), lambda h: (h, 0, 0)),
            pl.BlockSpec((1, L, DH), lambda h: (h, 0, 0)),
        ],
        out_shape=[
            jax.ShapeDtypeStruct((L, H), jnp.float32),
            jax.ShapeDtypeStruct((NH, NSEL * BS, 1), jnp.int32),
            jax.ShapeDtypeStruct((NH, L, DH), jnp.float32),
            jax.ShapeDtypeStruct((NH, L, DH), jnp.float32),
        ],
        compiler_params=pltpu.CompilerParams(
            dimension_semantics=("parallel",)),
    )(y, y, y)


NROW = NSEL * BS
NCH = NROW // 128


def _sc_gather_body(ktab, vtab, idx_hbm, out_hbm, idx_v, rows_v, sem):
    c = lax.axis_index("c")
    s = lax.axis_index("s")
    pltpu.sync_copy(idx_hbm.at[s], idx_v)

    @pl.when(c == 0)
    def _():
        cps = [pltpu.async_copy(ktab.at[idx_v.at[j]],
                                rows_v.at[pl.ds(j * 128, 128)], sem)
               for j in range(NCH)]
        for cp in cps:
            cp.wait()

    @pl.when(c == 1)
    def _():
        cps = [pltpu.async_copy(vtab.at[idx_v.at[j]],
                                rows_v.at[pl.ds(j * 128, 128)], sem)
               for j in range(NCH)]
        for cp in cps:
            cp.wait()

    pltpu.sync_copy(rows_v, out_hbm.at[c * NH + s])


def _sc_gather(ktab, vtab, idx3):
    mesh = plsc.VectorSubcoreMesh(core_axis_name="c", subcore_axis_name="s")
    fn = pl.kernel(
        _sc_gather_body,
        out_type=jax.ShapeDtypeStruct((2 * NH, NROW, DH), jnp.float32),
        mesh=mesh,
        scratch_types=[
            pltpu.VMEM((NCH, 128), jnp.int32),
            pltpu.VMEM((NROW, DH), jnp.float32),
            pltpu.SemaphoreType.DMA,
        ],
    )
    return fn(ktab, vtab, idx3)


def _attn_body(q_ref, kc_ref, kp_ref, vc_ref, vp_ref, sk_ref, sv_ref,
               comp_ref, gate_ref, o_ref):
    qi = pl.program_id(1)
    q = q_ref[...]

    sk = sk_ref[0]
    sv = sv_ref[0]
    ss = lax.dot_general(q, sk, (((1,), (1,)), ((), ())),
                         preferred_element_type=jnp.float32) * SCALE
    ms = jnp.max(ss, axis=-1, keepdims=True)
    es = jnp.exp(ss - ms)
    sel_out = lax.dot_general(es, sv, (((1,), (0,)), ((), ())),
                              preferred_element_type=jnp.float32)
    sel_out = sel_out / jnp.sum(es, axis=-1, keepdims=True)

    i_ = lax.broadcasted_iota(jnp.int32, (QT, QT), 0)
    j_ = lax.broadcasted_iota(jnp.int32, (QT, QT), 1)
    kc = kc_ref[0]
    vc = vc_ref[0]
    kp = kp_ref[0]
    vp = vp_ref[0]
    sc = lax.dot_general(q, kc, (((1,), (1,)), ((), ())),
                         preferred_element_type=jnp.float32) * SCALE
    sc = jnp.where(i_ >= j_, sc, NEG)
    sp = lax.dot_general(q, kp, (((1,), (1,)), ((), ())),
                         preferred_element_type=jnp.float32) * SCALE
    sp = jnp.where((j_ > i_) & (qi > 0), sp, NEG)
    m = jnp.maximum(jnp.max(sc, axis=-1, keepdims=True),
                    jnp.max(sp, axis=-1, keepdims=True))
    ec = jnp.exp(sc - m)
    ep = jnp.exp(sp - m)
    den = jnp.sum(ec, axis=-1, keepdims=True) + jnp.sum(ep, axis=-1,
                                                        keepdims=True)
    sl_out = (lax.dot_general(ec, vc, (((1,), (0,)), ((), ())),
                              preferred_element_type=jnp.float32)
              + lax.dot_general(ep, vp, (((1,), (0,)), ((), ())),
                                preferred_element_type=jnp.float32)) / den

    g = jax.nn.sigmoid(gate_ref[...])
    o_ref[...] = (g[:, 0:1] * comp_ref[...]
                  + g[:, 1:2] * sel_out
                  + g[:, 2:3] * sl_out)


def _attend_combine(y, khead, vhead, skv, comp):
    return pl.pallas_call(
        _attn_body,
        grid=(NH, NQT),
        in_specs=[
            pl.BlockSpec((QT, DH), lambda h, qi: (qi, h)),
            pl.BlockSpec((1, QT, DH), lambda h, qi: (h, qi, 0)),
            pl.BlockSpec((1, QT, DH),
                         lambda h, qi: (h, jnp.maximum(qi - 1, 0), 0)),
            pl.BlockSpec((1, QT, DH), lambda h, qi: (h, qi, 0)),
            pl.BlockSpec((1, QT, DH),
                         lambda h, qi: (h, jnp.maximum(qi - 1, 0), 0)),
            pl.BlockSpec((1, NROW, DH), lambda h, qi: (h, 0, 0)),
            pl.BlockSpec((1, NROW, DH), lambda h, qi: (NH + h, 0, 0)),
            pl.BlockSpec((QT, DH), lambda h, qi: (qi, h)),
            pl.BlockSpec((QT, 128), lambda h, qi: (qi, 3 * NH)),
        ],
        out_specs=pl.BlockSpec((QT, DH), lambda h, qi: (qi, h)),
        out_shape=jax.ShapeDtypeStruct((L, H), jnp.float32),
        compiler_params=pltpu.CompilerParams(
            dimension_semantics=("parallel", "arbitrary")),
    )(y, khead, khead, vhead, vhead, skv, skv, comp, y)


def _out_proj(attn, wo, bo):
    return pl.pallas_call(
        _proj_body,
        grid=(L // 512, H // 512),
        in_specs=[
            pl.BlockSpec((512, H), lambda i, j: (i, 0)),
            pl.BlockSpec((H, 512), lambda i, j: (0, j)),
            pl.BlockSpec((8, 512), lambda i, j: (0, j)),
        ],
        out_specs=pl.BlockSpec((512, 512), lambda i, j: (i, j)),
        out_shape=jax.ShapeDtypeStruct((L, H), jnp.float32),
        compiler_params=pltpu.CompilerParams(
            dimension_semantics=("parallel", "parallel")),
    )(attn, wo, jnp.broadcast_to(bo, (8, H)))


@jax.jit
def kernel(hidden_states, Wq, bq, Wk, bk, Wv, bv, Wo, bo, Wg, bg):
    hs = hidden_states.reshape(L, H)
    wg_pad = jnp.zeros((H, 128), jnp.float32).at[:, :3].set(Wg)
    bg_pad = jnp.zeros((128,), jnp.float32).at[:3].set(bg)
    wall = jnp.concatenate([Wq, Wk, Wv, wg_pad], axis=1)
    ball = jnp.concatenate([bq, bk, bv, bg_pad])

    y = _projection(hs, wall, ball)
    comp, idx, khead, vhead = _compress_select(y)
    skv = _sc_gather(khead.reshape(NH * L, DH),
                     vhead.reshape(NH * L, DH),
                     idx.reshape(NH, NCH, 128))
    attn = _attend_combine(y, khead, vhead, skv, comp)
    out = _out_proj(attn, Wo, bo)
    return out.reshape(B_, L, H)
